# Initial kernel scaffold; baseline (speedup 1.0000x reference)
import jax
import jax.numpy as jnp
from jax.experimental import pallas as pl


def kernel(logits, targets):
    raise NotImplementedError("write your pallas kernel here")


# speedup vs baseline: 1.5945x; 1.5945x over previous
import functools

import jax
import jax.numpy as jnp
from jax import lax
from jax.experimental import pallas as pl

_IGNORE = 255
_THRESH = 0.7
_MIN_KEPT = 100000


def _ce_body(x_ref, t_ref, loss_ref, mp_ref):
    x = x_ref[0]
    t = t_ref[0]
    m = jnp.max(x, axis=0, keepdims=True)
    sh = x - m
    s = jnp.sum(jnp.exp(sh), axis=0, keepdims=True)
    cls = lax.broadcasted_iota(jnp.int32, x.shape, 0)
    sh_t = jnp.sum(jnp.where(cls == t, sh, 0.0), axis=0, keepdims=True)
    loss = jnp.log(s) - sh_t
    valid = t != _IGNORE
    loss_ref[0] = jnp.where(valid, loss, -1.0)
    mp_ref[0] = jnp.where(valid, 1.0 / s, 2.0)


def _sel_body(loss_ref, mp_ref, out_ref):
    loss = loss_ref[...]
    mp = mp_ref[...]
    npix = jnp.sum((loss >= 0.0).astype(jnp.int32))
    nhard = jnp.sum((mp < _THRESH).astype(jnp.int32))
    min_kept = jnp.minimum(_MIN_KEPT, npix)
    k = jnp.minimum(jnp.maximum(min_kept, nhard), npix)
    bits = lax.bitcast_convert_type(loss, jnp.int32)

    def step(_, lohi):
        lo, hi = lohi
        mid = lo + lax.div(hi - lo, 2)
        cnt = jnp.sum((bits >= mid).astype(jnp.int32))
        big = cnt >= k
        return jnp.where(big, mid, lo), jnp.where(big, hi, mid)

    lo, _ = lax.fori_loop(0, 31, step, (jnp.int32(0), jnp.int32(0x7F800000)))
    thresh = lax.bitcast_convert_type(lo, jnp.float32)
    keep = loss >= thresh
    cnt = jnp.sum(keep.astype(jnp.int32))
    hsum = jnp.sum(jnp.where(keep, loss, 0.0))
    mean = hsum / jnp.maximum(cnt, 1).astype(jnp.float32)
    out_ref[...] = jnp.where(npix == 0, 0.0, mean).reshape(1, 1)


@functools.partial(jax.jit, static_argnames=("tile",))
def _run(logits, targets, tile=1792):
    B, C, H, W = logits.shape
    HW = H * W
    n = HW // tile
    x3 = logits.reshape(B, C, HW)
    t3 = targets.reshape(B, 1, HW)
    loss, mp = pl.pallas_call(
        _ce_body,
        grid=(B, n),
        in_specs=[
            pl.BlockSpec((1, C, tile), lambda b, j: (b, 0, j)),
            pl.BlockSpec((1, 1, tile), lambda b, j: (b, 0, j)),
        ],
        out_specs=[
            pl.BlockSpec((1, 1, tile), lambda b, j: (b, 0, j)),
            pl.BlockSpec((1, 1, tile), lambda b, j: (b, 0, j)),
        ],
        out_shape=[
            jax.ShapeDtypeStruct((B, 1, HW), jnp.float32),
            jax.ShapeDtypeStruct((B, 1, HW), jnp.float32),
        ],
    )(x3, t3)
    rows = (B * HW) // 128
    out = pl.pallas_call(
        _sel_body,
        out_shape=jax.ShapeDtypeStruct((1, 1), jnp.float32),
    )(loss.reshape(rows, 128), mp.reshape(rows, 128))
    return out[0, 0]


def kernel(logits, targets):
    return _run(logits, targets)


# speedup vs baseline: 6.9804x; 4.3778x over previous
import functools

import jax
import jax.numpy as jnp
from jax import lax
from jax.experimental import pallas as pl

_IGNORE = 255
_THRESH = 0.7
_MIN_KEPT = 100000


def _ce_body(x_ref, t_ref, loss_ref, mp_ref):
    x = x_ref[0]
    t = t_ref[0]
    m = jnp.max(x, axis=0)
    sh = x - m[None]
    s = jnp.sum(jnp.exp(sh), axis=0)
    cls = lax.broadcasted_iota(jnp.int32, x.shape, 0)
    sh_t = jnp.sum(jnp.where(cls == t[None], sh, 0.0), axis=0)
    loss = jnp.log(s) - sh_t
    valid = t != _IGNORE
    loss_ref[0] = jnp.where(valid, loss, -1.0)
    mp_ref[0] = jnp.where(valid, 1.0 / s, 2.0)


def _sel_body(loss_ref, mp_ref, out_ref):
    loss = loss_ref[...]
    mp = mp_ref[...]
    npix = jnp.sum((loss >= 0.0).astype(jnp.int32))
    nhard = jnp.sum((mp < _THRESH).astype(jnp.int32))
    min_kept = jnp.minimum(_MIN_KEPT, npix)
    k = jnp.minimum(jnp.maximum(min_kept, nhard), npix)
    bits = lax.bitcast_convert_type(loss, jnp.int32)

    def step(_, lohi):
        lo, hi = lohi
        mid = lo + lax.div(hi - lo, 2)
        cnt = jnp.sum((bits >= mid).astype(jnp.int32))
        big = cnt >= k
        return jnp.where(big, mid, lo), jnp.where(big, hi, mid)

    lo, _ = lax.fori_loop(0, 31, step, (jnp.int32(0), jnp.int32(0x7F800000)))
    thresh = lax.bitcast_convert_type(lo, jnp.float32)
    keep = loss >= thresh
    cnt = jnp.sum(keep.astype(jnp.int32))
    hsum = jnp.sum(jnp.where(keep, loss, 0.0))
    mean = hsum / jnp.maximum(cnt, 1).astype(jnp.float32)
    out_ref[...] = jnp.where(npix == 0, 0.0, mean).reshape(1, 1)


@functools.partial(jax.jit, static_argnames=("hb", "p1only"))
def _run(logits, targets, hb=32, p1only=False):
    B, C, H, W = logits.shape
    nh = H // hb
    loss, mp = pl.pallas_call(
        _ce_body,
        grid=(B, nh),
        in_specs=[
            pl.BlockSpec((1, C, hb, W), lambda b, h: (b, 0, h, 0)),
            pl.BlockSpec((1, hb, W), lambda b, h: (b, h, 0)),
        ],
        out_specs=[
            pl.BlockSpec((1, hb, W), lambda b, h: (b, h, 0)),
            pl.BlockSpec((1, hb, W), lambda b, h: (b, h, 0)),
        ],
        out_shape=[
            jax.ShapeDtypeStruct((B, H, W), jnp.float32),
            jax.ShapeDtypeStruct((B, H, W), jnp.float32),
        ],
    )(logits, targets)
    if p1only:
        return loss[0, 0, 0] + mp[0, 0, 0]
    out = pl.pallas_call(
        _sel_body,
        out_shape=jax.ShapeDtypeStruct((1, 1), jnp.float32),
    )(loss, mp)
    return out[0, 0]


def kernel(logits, targets):
    return _run(logits, targets)
